# baseline (speedup 1.0000x reference)
import functools

import jax
import jax.numpy as jnp
from jax import lax
from jax.experimental import pallas as pl
from jax.experimental.pallas import tpu as pltpu
from jax.experimental.pallas import tpu_sc as plsc

N = 10000
E = 160000
HID = 256
D = 512
NPAD = 10240
CHUNK = 160
NCHUNK = 64
NW = 32
NPASS = 2
EP = E + NCHUNK * 16 + 2048


def _make_agg():
    mesh = plsc.VectorSubcoreMesh(core_axis_name="c", subcore_axis_name="s")

    @functools.partial(
        pl.kernel,
        mesh=mesh,
        out_type=jax.ShapeDtypeStruct((NPAD, D), jnp.float32),
        scratch_types=[
            pltpu.VMEM((CHUNK, D), jnp.float32),
            pltpu.VMEM((NCHUNK * 8 + 16,), jnp.int32),
            pltpu.VMEM((2048,), jnp.int32),
            pltpu.VMEM((2048,), jnp.int32),
            pltpu.VMEM((2048,), jnp.float32),
            pltpu.VMEM((2048,), jnp.float32),
            pltpu.VMEM((16, D), jnp.float32),
            pltpu.VMEM((16, D), jnp.float32),
            pltpu.SemaphoreType.DMA,
            pltpu.SemaphoreType.DMA,
        ],
    )
    def agg(h_hbm, srcs_hbm, dls_hbm, w0_hbm, w1_hbm, starts_hbm, out_hbm,
            acc, sv, esrc, edl, ew0, ew1, stage0, stage1, sem0, sem1):
        wid = lax.axis_index("s") * 2 + lax.axis_index("c")
        zf = jnp.zeros((16,), jnp.float32)

        pltpu.sync_copy(starts_hbm, sv)

        for p in range(NPASS):
            c = p * NW + wid
            lo = pl.multiple_of(c * CHUNK, 32)

            def zrow(r, _):
                for j in range(32):
                    acc[r, pl.ds(j * 16, 16)] = zf
                return 0

            lax.fori_loop(0, CHUNK, zrow, 0)

            se = sv[pl.ds(pl.multiple_of(c * 8, 8), 16)]
            start = se[0]
            end = se[1]
            ngroups = (end - start) // 16

            stages = (stage0, stage1)
            sems = (sem0, sem1)

            nslabs = (ngroups + 127) // 128

            def slab(sb, _):
                gbase = sb * 128
                ng = jnp.minimum(ngroups - gbase, 128)
                ebase = pl.multiple_of(start + gbase * 16, 16)
                pltpu.sync_copy(srcs_hbm.at[pl.ds(ebase, 2048)], esrc)
                pltpu.sync_copy(dls_hbm.at[pl.ds(ebase, 2048)], edl)
                pltpu.sync_copy(w0_hbm.at[pl.ds(ebase, 2048)], ew0)
                pltpu.sync_copy(w1_hbm.at[pl.ds(ebase, 2048)], ew1)

                @pl.when(ng > 0)
                def _():
                    idx0 = esrc[pl.ds(0, 16)]
                    pltpu.make_async_copy(h_hbm.at[idx0], stage0, sem0).start()

                def pair(q, _):
                    for b in range(2):
                        g = q * 2 + b

                        @pl.when(g < ng)
                        def _():
                            idxg = esrc[pl.ds(g * 16, 16)]
                            pltpu.make_async_copy(
                                h_hbm.at[idxg], stages[b], sems[b]).wait()

                            @pl.when(g + 1 < ng)
                            def _():
                                idxn = esrc[pl.ds((g + 1) * 16, 16)]
                                pltpu.make_async_copy(
                                    h_hbm.at[idxn], stages[1 - b], sems[1 - b]).start()

                            dlv = edl[pl.ds(g * 16, 16)]
                            w0v = ew0[pl.ds(g * 16, 16)]
                            w1v = ew1[pl.ds(g * 16, 16)]
                            for k in range(16):
                                dl = dlv[k]
                                w0k = w0v[k]
                                w1k = w1v[k]

                                def jbody(jq, _, _k=k, _dl=dl, _w0=w0k, _w1=w1k, _b=b):
                                    off0 = pl.multiple_of(_k * 0 + jq * 64, 64)
                                    for u in range(4):
                                        off = off0 + u * 16
                                        w = jnp.where(jq * 64 + u * 16 < 256, _w0, _w1)
                                        acc[_dl, pl.ds(off, 16)] = (
                                            acc[_dl, pl.ds(off, 16)]
                                            + w * stages[_b][_k, pl.ds(off, 16)])
                                    return 0

                                lax.fori_loop(0, 8, jbody, 0)
                    return 0

                lax.fori_loop(0, (ng + 1) // 2, pair, 0)
                return 0

            lax.fori_loop(0, nslabs, slab, 0)
            pltpu.sync_copy(acc, out_hbm.at[pl.ds(lo, CHUNK)])

    return agg


def _make_msg2():
    mesh = plsc.VectorSubcoreMesh(core_axis_name="c", subcore_axis_name="s")
    NG = E // 16
    GPW = (NG + NW - 1) // NW

    @functools.partial(
        pl.kernel,
        mesh=mesh,
        out_type=jax.ShapeDtypeStruct((E, HID), jnp.float32),
        scratch_types=[
            pltpu.VMEM((2048,), jnp.int32),
            pltpu.VMEM((2048,), jnp.float32),
            pltpu.VMEM((16, HID), jnp.float32),
            pltpu.VMEM((16, HID), jnp.float32),
            pltpu.VMEM((16, HID), jnp.float32),
            pltpu.SemaphoreType.DMA,
            pltpu.SemaphoreType.DMA,
        ],
    )
    def msg2(h_hbm, src_hbm, w_hbm, out_hbm, esrc, ew, stage0, stage1, obuf,
             sem0, sem1):
        wid = lax.axis_index("s") * 2 + lax.axis_index("c")
        g0 = wid * GPW
        g1 = jnp.minimum(g0 + GPW, NG)
        ng = g1 - g0
        stages = (stage0, stage1)
        sems = (sem0, sem1)
        nslabs = (ng + 127) // 128

        def slab(sb, _):
            gbase = sb * 128
            ns = jnp.minimum(ng - gbase, 128)
            ebase = pl.multiple_of((g0 + gbase) * 16, 16)
            pltpu.sync_copy(src_hbm.at[pl.ds(ebase, 2048)], esrc)
            pltpu.sync_copy(w_hbm.at[pl.ds(ebase, 2048)], ew)

            @pl.when(ns > 0)
            def _():
                idx0 = esrc[pl.ds(0, 16)]
                pltpu.make_async_copy(h_hbm.at[idx0], stage0, sem0).start()

            def pair(q, _):
                for b in range(2):
                    g = q * 2 + b

                    @pl.when(g < ns)
                    def _():
                        idxg = esrc[pl.ds(g * 16, 16)]
                        pltpu.make_async_copy(
                            h_hbm.at[idxg], stages[b], sems[b]).wait()

                        @pl.when(g + 1 < ns)
                        def _():
                            idxn = esrc[pl.ds((g + 1) * 16, 16)]
                            pltpu.make_async_copy(
                                h_hbm.at[idxn], stages[1 - b], sems[1 - b]).start()

                        wv = ew[pl.ds(g * 16, 16)]
                        for k in range(16):
                            wk = wv[k]

                            def jbody(jq, _, _k=k, _w=wk, _b=b):
                                off = pl.multiple_of(jq * 64, 64)
                                for u in range(4):
                                    o = off + u * 16
                                    obuf[_k, pl.ds(o, 16)] = (
                                        _w * stages[_b][_k, pl.ds(o, 16)])
                                return 0

                            lax.fori_loop(0, 4, jbody, 0)
                        orow = pl.multiple_of((g0 + gbase + g) * 16, 16)
                        pltpu.sync_copy(obuf, out_hbm.at[pl.ds(orow, 16)])
                return 0

            lax.fori_loop(0, (ns + 1) // 2, pair, 0)
            return 0

        lax.fori_loop(0, nslabs, slab, 0)

    return msg2


def _edge_prep(src, dst):
    perm = jnp.argsort(dst, stable=True)
    dsts = dst[perm]
    first = jnp.searchsorted(
        dsts, jnp.arange(NCHUNK + 1, dtype=jnp.int32) * CHUNK).astype(jnp.int32)
    cnt = first[1:] - first[:-1]
    pcnt = (cnt + 15) // 16 * 16
    sa = jnp.concatenate(
        [jnp.zeros((1,), jnp.int32), jnp.cumsum(pcnt).astype(jnp.int32)])
    slot = jnp.arange(EP, dtype=jnp.int32)
    ci = jnp.searchsorted(sa, slot, side="right").astype(jnp.int32) - 1
    ci = jnp.clip(ci, 0, NCHUNK - 1)
    off = slot - sa[ci]
    valid = off < cnt[ci]
    sidx = jnp.clip(first[ci] + off, 0, E - 1)
    eidx = perm[sidx]
    srcs_p = jnp.where(valid, src[eidx], 0)
    dls_p = jnp.where(valid, dst[eidx] % CHUNK, 0)
    idx8 = jnp.arange(NCHUNK, dtype=jnp.int32) * 8
    s8 = jnp.zeros((NCHUNK * 8 + 16,), jnp.int32)
    s8 = s8.at[idx8].set(sa[:-1]).at[idx8 + 1].set(sa[1:])
    return eidx, valid, srcs_p, dls_p, s8


def _batch_norm(x, gamma, beta):
    mu = x.mean(axis=0, keepdims=True)
    var = x.var(axis=0, keepdims=True)
    return (x - mu) / jnp.sqrt(var + 1e-5) * gamma + beta


def kernel(x, edge_index, W1, att_src1, att_dst1, b1, g1, be1,
           W2, att_src2, att_dst2, b2, g2, be2):
    src = edge_index[0].astype(jnp.int32)
    dst = edge_index[1].astype(jnp.int32)

    eidx, valid, srcs_p, dls_p, s8 = _edge_prep(src, dst)
    sc_agg = _make_agg()

    h = (x @ W1).reshape(N, 2, HID)
    alpha_s = jnp.sum(h * att_src1[None, :, :], axis=-1)
    alpha_d = jnp.sum(h * att_dst1[None, :, :], axis=-1)
    e = jax.nn.leaky_relu(alpha_s[src] + alpha_d[dst], negative_slope=0.2)
    m = jax.ops.segment_max(e, dst, num_segments=N)
    ex = jnp.exp(e - m[dst])
    s = jax.ops.segment_sum(ex, dst, num_segments=N)
    alpha = ex / (s[dst] + 1e-16)
    w0_p = jnp.where(valid, alpha[eidx, 0], 0.0)
    w1_p = jnp.where(valid, alpha[eidx, 1], 0.0)
    out1 = sc_agg(h.reshape(N, D), srcs_p, dls_p, w0_p, w1_p, s8)[:N]
    h1 = out1 + b1
    h1 = _batch_norm(h1, g1, be1)
    h1 = jax.nn.relu(h1)

    h2 = (h1 @ W2).reshape(N, 1, HID)
    alpha_s2 = jnp.sum(h2 * att_src2[None, :, :], axis=-1)
    alpha_d2 = jnp.sum(h2 * att_dst2[None, :, :], axis=-1)
    e2 = jax.nn.leaky_relu(alpha_s2[src] + alpha_d2[dst], negative_slope=0.2)
    m2 = jax.ops.segment_max(e2, dst, num_segments=N)
    ex2 = jnp.exp(e2 - m2[dst])
    s2 = jax.ops.segment_sum(ex2, dst, num_segments=N)
    alpha2 = ex2 / (s2[dst] + 1e-16)
    src_pad = jnp.concatenate([src, jnp.zeros((2048,), jnp.int32)])
    w2_pad = jnp.concatenate([alpha2[:, 0], jnp.zeros((2048,), jnp.float32)])
    msg2 = _make_msg2()(h2.reshape(N, HID), src_pad, w2_pad)
    out2 = jax.ops.segment_sum(
        msg2.reshape(E, 1, HID), dst, num_segments=N)
    h2o = out2.mean(axis=1) + b2
    h2b = _batch_norm(h2o, g2, be2)
    return h2b.mean(axis=0, keepdims=True)


# speedup vs baseline: 1.3225x; 1.3225x over previous
import functools

import jax
import jax.numpy as jnp
from jax import lax
from jax.experimental import pallas as pl
from jax.experimental.pallas import tpu as pltpu
from jax.experimental.pallas import tpu_sc as plsc

N = 10000
E = 160000
HID = 256
D = 512
NPAD = 10240
CHUNK = 160
NCHUNK = 64
NW = 32
NPASS = 2
EP = E + NCHUNK * 16 + 2048


def _make_agg():
    mesh = plsc.VectorSubcoreMesh(core_axis_name="c", subcore_axis_name="s")

    @functools.partial(
        pl.kernel,
        mesh=mesh,
        out_type=jax.ShapeDtypeStruct((NPAD, D), jnp.float32),
        scratch_types=[
            pltpu.VMEM((CHUNK, D), jnp.float32),
            pltpu.VMEM((NCHUNK * 8 + 16,), jnp.int32),
            pltpu.VMEM((2048,), jnp.int32),
            pltpu.VMEM((2048,), jnp.int32),
            pltpu.VMEM((2048,), jnp.float32),
            pltpu.VMEM((2048,), jnp.float32),
            pltpu.VMEM((16, D), jnp.float32),
            pltpu.VMEM((16, D), jnp.float32),
            pltpu.SemaphoreType.DMA,
            pltpu.SemaphoreType.DMA,
        ],
    )
    def agg(h_hbm, srcs_hbm, dls_hbm, w0_hbm, w1_hbm, starts_hbm, out_hbm,
            acc, sv, esrc, edl, ew0, ew1, stage0, stage1, sem0, sem1):
        wid = lax.axis_index("s") * 2 + lax.axis_index("c")
        zf = jnp.zeros((16,), jnp.float32)

        pltpu.sync_copy(starts_hbm, sv)

        for p in range(NPASS):
            c = p * NW + wid
            lo = pl.multiple_of(c * CHUNK, 32)

            def zrow(r, _):
                for j in range(32):
                    acc[r, pl.ds(j * 16, 16)] = zf
                return 0

            lax.fori_loop(0, CHUNK, zrow, 0)

            se = sv[pl.ds(pl.multiple_of(c * 8, 8), 16)]
            start = se[0]
            end = se[1]
            ngroups = (end - start) // 16

            stages = (stage0, stage1)
            sems = (sem0, sem1)

            nslabs = (ngroups + 127) // 128

            def slab(sb, _):
                gbase = sb * 128
                ng = jnp.minimum(ngroups - gbase, 128)
                ebase = pl.multiple_of(start + gbase * 16, 16)
                pltpu.sync_copy(srcs_hbm.at[pl.ds(ebase, 2048)], esrc)
                pltpu.sync_copy(dls_hbm.at[pl.ds(ebase, 2048)], edl)
                pltpu.sync_copy(w0_hbm.at[pl.ds(ebase, 2048)], ew0)
                pltpu.sync_copy(w1_hbm.at[pl.ds(ebase, 2048)], ew1)

                @pl.when(ng > 0)
                def _():
                    idx0 = esrc[pl.ds(0, 16)]
                    pltpu.make_async_copy(h_hbm.at[idx0], stage0, sem0).start()

                def pair(q, _):
                    for b in range(2):
                        g = q * 2 + b

                        @pl.when(g < ng)
                        def _():
                            idxg = esrc[pl.ds(g * 16, 16)]
                            pltpu.make_async_copy(
                                h_hbm.at[idxg], stages[b], sems[b]).wait()

                            @pl.when(g + 1 < ng)
                            def _():
                                idxn = esrc[pl.ds((g + 1) * 16, 16)]
                                pltpu.make_async_copy(
                                    h_hbm.at[idxn], stages[1 - b], sems[1 - b]).start()

                            dlv = edl[pl.ds(g * 16, 16)]
                            w0v = ew0[pl.ds(g * 16, 16)]
                            w1v = ew1[pl.ds(g * 16, 16)]
                            for k in range(16):
                                dl = dlv[k]
                                w0k = w0v[k]
                                w1k = w1v[k]

                                def jbody(jq, _, _k=k, _dl=dl, _w0=w0k, _w1=w1k, _b=b):
                                    off0 = pl.multiple_of(_k * 0 + jq * 64, 64)
                                    for u in range(4):
                                        off = off0 + u * 16
                                        w = jnp.where(jq * 64 + u * 16 < 256, _w0, _w1)
                                        acc[_dl, pl.ds(off, 16)] = (
                                            acc[_dl, pl.ds(off, 16)]
                                            + w * stages[_b][_k, pl.ds(off, 16)])
                                    return 0

                                lax.fori_loop(0, 8, jbody, 0)
                    return 0

                lax.fori_loop(0, (ng + 1) // 2, pair, 0)
                return 0

            lax.fori_loop(0, nslabs, slab, 0)
            pltpu.sync_copy(acc, out_hbm.at[pl.ds(lo, CHUNK)])

    return agg


def _make_msg2():
    mesh = plsc.VectorSubcoreMesh(core_axis_name="c", subcore_axis_name="s")
    NG = E // 16
    GPW = (NG + NW - 1) // NW

    @functools.partial(
        pl.kernel,
        mesh=mesh,
        out_type=jax.ShapeDtypeStruct((E, HID), jnp.float32),
        scratch_types=[
            pltpu.VMEM((2048,), jnp.int32),
            pltpu.VMEM((2048,), jnp.float32),
            pltpu.VMEM((16, HID), jnp.float32),
            pltpu.VMEM((16, HID), jnp.float32),
            pltpu.VMEM((16, HID), jnp.float32),
            pltpu.SemaphoreType.DMA,
            pltpu.SemaphoreType.DMA,
        ],
    )
    def msg2(h_hbm, src_hbm, w_hbm, out_hbm, esrc, ew, stage0, stage1, obuf,
             sem0, sem1):
        wid = lax.axis_index("s") * 2 + lax.axis_index("c")
        g0 = wid * GPW
        g1 = jnp.minimum(g0 + GPW, NG)
        ng = g1 - g0
        stages = (stage0, stage1)
        sems = (sem0, sem1)
        nslabs = (ng + 127) // 128

        def slab(sb, _):
            gbase = sb * 128
            ns = jnp.minimum(ng - gbase, 128)
            ebase = pl.multiple_of((g0 + gbase) * 16, 16)
            pltpu.sync_copy(src_hbm.at[pl.ds(ebase, 2048)], esrc)
            pltpu.sync_copy(w_hbm.at[pl.ds(ebase, 2048)], ew)

            @pl.when(ns > 0)
            def _():
                idx0 = esrc[pl.ds(0, 16)]
                pltpu.make_async_copy(h_hbm.at[idx0], stage0, sem0).start()

            def pair(q, _):
                for b in range(2):
                    g = q * 2 + b

                    @pl.when(g < ns)
                    def _():
                        idxg = esrc[pl.ds(g * 16, 16)]
                        pltpu.make_async_copy(
                            h_hbm.at[idxg], stages[b], sems[b]).wait()

                        @pl.when(g + 1 < ns)
                        def _():
                            idxn = esrc[pl.ds((g + 1) * 16, 16)]
                            pltpu.make_async_copy(
                                h_hbm.at[idxn], stages[1 - b], sems[1 - b]).start()

                        wv = ew[pl.ds(g * 16, 16)]
                        for k in range(16):
                            wk = wv[k]

                            def jbody(jq, _, _k=k, _w=wk, _b=b):
                                off = pl.multiple_of(jq * 64, 64)
                                for u in range(4):
                                    o = off + u * 16
                                    obuf[_k, pl.ds(o, 16)] = (
                                        _w * stages[_b][_k, pl.ds(o, 16)])
                                return 0

                            lax.fori_loop(0, 4, jbody, 0)
                        orow = pl.multiple_of((g0 + gbase + g) * 16, 16)
                        pltpu.sync_copy(obuf, out_hbm.at[pl.ds(orow, 16)])
                return 0

            lax.fori_loop(0, (ns + 1) // 2, pair, 0)
            return 0

        lax.fori_loop(0, nslabs, slab, 0)

    return msg2


def _edge_prep(src, dst):
    perm = jnp.argsort(dst, stable=True)
    dsts = dst[perm]
    first = jnp.searchsorted(
        dsts, jnp.arange(NCHUNK + 1, dtype=jnp.int32) * CHUNK).astype(jnp.int32)
    cnt = first[1:] - first[:-1]
    pcnt = (cnt + 15) // 16 * 16
    sa = jnp.concatenate(
        [jnp.zeros((1,), jnp.int32), jnp.cumsum(pcnt).astype(jnp.int32)])
    ci = dsts // CHUNK
    pos = sa[ci] + (jnp.arange(E, dtype=jnp.int32) - first[ci])
    srcs_p = jnp.zeros((EP,), jnp.int32).at[pos].set(
        src[perm], unique_indices=True)
    dls_p = jnp.zeros((EP,), jnp.int32).at[pos].set(
        dsts % CHUNK, unique_indices=True)
    idx8 = jnp.arange(NCHUNK, dtype=jnp.int32) * 8
    s8 = jnp.zeros((NCHUNK * 8 + 16,), jnp.int32)
    s8 = s8.at[idx8].set(sa[:-1]).at[idx8 + 1].set(sa[1:])
    return perm, pos, srcs_p, dls_p, s8


def _batch_norm(x, gamma, beta):
    mu = x.mean(axis=0, keepdims=True)
    var = x.var(axis=0, keepdims=True)
    return (x - mu) / jnp.sqrt(var + 1e-5) * gamma + beta


def kernel(x, edge_index, W1, att_src1, att_dst1, b1, g1, be1,
           W2, att_src2, att_dst2, b2, g2, be2):
    src = edge_index[0].astype(jnp.int32)
    dst = edge_index[1].astype(jnp.int32)

    perm, pos, srcs_p, dls_p, s8 = _edge_prep(src, dst)
    sc_agg = _make_agg()

    h = (x @ W1).reshape(N, 2, HID)
    alpha_s = jnp.sum(h * att_src1[None, :, :], axis=-1)
    alpha_d = jnp.sum(h * att_dst1[None, :, :], axis=-1)
    e = jax.nn.leaky_relu(alpha_s[src] + alpha_d[dst], negative_slope=0.2)
    m = jax.ops.segment_max(e, dst, num_segments=N)
    ex = jnp.exp(e - m[dst])
    s = jax.ops.segment_sum(ex, dst, num_segments=N)
    alpha = ex / (s[dst] + 1e-16)
    w0_p = jnp.zeros((EP,), jnp.float32).at[pos].set(
        alpha[perm, 0], unique_indices=True)
    w1_p = jnp.zeros((EP,), jnp.float32).at[pos].set(
        alpha[perm, 1], unique_indices=True)
    out1 = sc_agg(h.reshape(N, D), srcs_p, dls_p, w0_p, w1_p, s8)[:N]
    h1 = out1 + b1
    h1 = _batch_norm(h1, g1, be1)
    h1 = jax.nn.relu(h1)

    h2 = (h1 @ W2).reshape(N, 1, HID)
    alpha_s2 = jnp.sum(h2 * att_src2[None, :, :], axis=-1)
    alpha_d2 = jnp.sum(h2 * att_dst2[None, :, :], axis=-1)
    e2 = jax.nn.leaky_relu(alpha_s2[src] + alpha_d2[dst], negative_slope=0.2)
    m2 = jax.ops.segment_max(e2, dst, num_segments=N)
    ex2 = jnp.exp(e2 - m2[dst])
    s2 = jax.ops.segment_sum(ex2, dst, num_segments=N)
    alpha2 = ex2 / (s2[dst] + 1e-16)
    src_pad = jnp.concatenate([src, jnp.zeros((2048,), jnp.int32)])
    w2_pad = jnp.concatenate([alpha2[:, 0], jnp.zeros((2048,), jnp.float32)])
    msg2 = _make_msg2()(h2.reshape(N, HID), src_pad, w2_pad)
    out2 = jax.ops.segment_sum(
        msg2.reshape(E, 1, HID), dst, num_segments=N)
    h2o = out2.mean(axis=1) + b2
    h2b = _batch_norm(h2o, g2, be2)
    return h2b.mean(axis=0, keepdims=True)


# speedup vs baseline: 1.4821x; 1.1207x over previous
import functools

import jax
import jax.numpy as jnp
from jax import lax
from jax.experimental import pallas as pl
from jax.experimental.pallas import tpu as pltpu
from jax.experimental.pallas import tpu_sc as plsc

N = 10000
E = 160000
HID = 256
D = 512
NPAD = 10240
CHUNK = 160
NCHUNK = 64
NW = 32
NPASS = 2
EP = E + NCHUNK * 16 + 2048


def _make_agg():
    mesh = plsc.VectorSubcoreMesh(core_axis_name="c", subcore_axis_name="s")

    @functools.partial(
        pl.kernel,
        mesh=mesh,
        out_type=jax.ShapeDtypeStruct((NPAD, D), jnp.float32),
        scratch_types=[
            pltpu.VMEM((CHUNK, D), jnp.float32),
            pltpu.VMEM((NCHUNK * 8 + 16,), jnp.int32),
            pltpu.VMEM((2048,), jnp.int32),
            pltpu.VMEM((2048,), jnp.int32),
            pltpu.VMEM((2048,), jnp.float32),
            pltpu.VMEM((2048,), jnp.float32),
            pltpu.VMEM((16, D), jnp.float32),
            pltpu.VMEM((16, D), jnp.float32),
            pltpu.SemaphoreType.DMA,
            pltpu.SemaphoreType.DMA,
        ],
    )
    def agg(h_hbm, srcs_hbm, dls_hbm, w0_hbm, w1_hbm, starts_hbm, out_hbm,
            acc, sv, esrc, edl, ew0, ew1, stage0, stage1, sem0, sem1):
        wid = lax.axis_index("s") * 2 + lax.axis_index("c")
        zf = jnp.zeros((16,), jnp.float32)

        pltpu.sync_copy(starts_hbm, sv)

        for p in range(NPASS):
            c = p * NW + wid
            lo = pl.multiple_of(c * CHUNK, 32)

            def zrow(r, _):
                for j in range(32):
                    acc[r, pl.ds(j * 16, 16)] = zf
                return 0

            lax.fori_loop(0, CHUNK, zrow, 0)

            se = sv[pl.ds(pl.multiple_of(c * 8, 8), 16)]
            start = se[0]
            end = se[1]
            ngroups = (end - start) // 16

            stages = (stage0, stage1)
            sems = (sem0, sem1)

            nslabs = (ngroups + 127) // 128

            def slab(sb, _):
                gbase = sb * 128
                ng = jnp.minimum(ngroups - gbase, 128)
                ebase = pl.multiple_of(start + gbase * 16, 16)
                pltpu.sync_copy(srcs_hbm.at[pl.ds(ebase, 2048)], esrc)
                pltpu.sync_copy(dls_hbm.at[pl.ds(ebase, 2048)], edl)
                pltpu.sync_copy(w0_hbm.at[pl.ds(ebase, 2048)], ew0)
                pltpu.sync_copy(w1_hbm.at[pl.ds(ebase, 2048)], ew1)

                @pl.when(ng > 0)
                def _():
                    idx0 = esrc[pl.ds(0, 16)]
                    pltpu.make_async_copy(h_hbm.at[idx0], stage0, sem0).start()

                def pair(q, _):
                    for b in range(2):
                        g = q * 2 + b

                        @pl.when(g < ng)
                        def _():
                            idxg = esrc[pl.ds(g * 16, 16)]
                            pltpu.make_async_copy(
                                h_hbm.at[idxg], stages[b], sems[b]).wait()

                            @pl.when(g + 1 < ng)
                            def _():
                                idxn = esrc[pl.ds((g + 1) * 16, 16)]
                                pltpu.make_async_copy(
                                    h_hbm.at[idxn], stages[1 - b], sems[1 - b]).start()

                            dlv = edl[pl.ds(g * 16, 16)]
                            w0v = ew0[pl.ds(g * 16, 16)]
                            w1v = ew1[pl.ds(g * 16, 16)]
                            for k in range(16):
                                dl = dlv[k]
                                w0k = w0v[k]
                                w1k = w1v[k]

                                def jbody(jq, _, _k=k, _dl=dl, _w0=w0k, _w1=w1k, _b=b):
                                    off0 = pl.multiple_of(_k * 0 + jq * 64, 64)
                                    for u in range(4):
                                        off = off0 + u * 16
                                        w = jnp.where(jq * 64 + u * 16 < 256, _w0, _w1)
                                        acc[_dl, pl.ds(off, 16)] = (
                                            acc[_dl, pl.ds(off, 16)]
                                            + w * stages[_b][_k, pl.ds(off, 16)])
                                    return 0

                                lax.fori_loop(0, 8, jbody, 0)
                    return 0

                lax.fori_loop(0, (ng + 1) // 2, pair, 0)
                return 0

            lax.fori_loop(0, nslabs, slab, 0)
            pltpu.sync_copy(acc, out_hbm.at[pl.ds(lo, CHUNK)])

    return agg


def _make_msg2():
    mesh = plsc.VectorSubcoreMesh(core_axis_name="c", subcore_axis_name="s")
    NG = E // 16
    GPW = (NG + NW - 1) // NW

    @functools.partial(
        pl.kernel,
        mesh=mesh,
        out_type=jax.ShapeDtypeStruct((E, HID), jnp.float32),
        scratch_types=[
            pltpu.VMEM((2048,), jnp.int32),
            pltpu.VMEM((2048,), jnp.float32),
            pltpu.VMEM((16, HID), jnp.float32),
            pltpu.VMEM((16, HID), jnp.float32),
            pltpu.VMEM((16, HID), jnp.float32),
            pltpu.SemaphoreType.DMA,
            pltpu.SemaphoreType.DMA,
        ],
    )
    def msg2(h_hbm, src_hbm, w_hbm, out_hbm, esrc, ew, stage0, stage1, obuf,
             sem0, sem1):
        wid = lax.axis_index("s") * 2 + lax.axis_index("c")
        g0 = wid * GPW
        g1 = jnp.minimum(g0 + GPW, NG)
        ng = g1 - g0
        stages = (stage0, stage1)
        sems = (sem0, sem1)
        nslabs = (ng + 127) // 128

        def slab(sb, _):
            gbase = sb * 128
            ns = jnp.minimum(ng - gbase, 128)
            ebase = pl.multiple_of((g0 + gbase) * 16, 16)
            pltpu.sync_copy(src_hbm.at[pl.ds(ebase, 2048)], esrc)
            pltpu.sync_copy(w_hbm.at[pl.ds(ebase, 2048)], ew)

            @pl.when(ns > 0)
            def _():
                idx0 = esrc[pl.ds(0, 16)]
                pltpu.make_async_copy(h_hbm.at[idx0], stage0, sem0).start()

            def pair(q, _):
                for b in range(2):
                    g = q * 2 + b

                    @pl.when(g < ns)
                    def _():
                        idxg = esrc[pl.ds(g * 16, 16)]
                        pltpu.make_async_copy(
                            h_hbm.at[idxg], stages[b], sems[b]).wait()

                        @pl.when(g + 1 < ns)
                        def _():
                            idxn = esrc[pl.ds((g + 1) * 16, 16)]
                            pltpu.make_async_copy(
                                h_hbm.at[idxn], stages[1 - b], sems[1 - b]).start()

                        wv = ew[pl.ds(g * 16, 16)]
                        for k in range(16):
                            wk = wv[k]

                            def jbody(jq, _, _k=k, _w=wk, _b=b):
                                off = pl.multiple_of(jq * 64, 64)
                                for u in range(4):
                                    o = off + u * 16
                                    obuf[_k, pl.ds(o, 16)] = (
                                        _w * stages[_b][_k, pl.ds(o, 16)])
                                return 0

                            lax.fori_loop(0, 4, jbody, 0)
                        orow = pl.multiple_of((g0 + gbase + g) * 16, 16)
                        pltpu.sync_copy(obuf, out_hbm.at[pl.ds(orow, 16)])
                return 0

            lax.fori_loop(0, (ns + 1) // 2, pair, 0)
            return 0

        lax.fori_loop(0, nslabs, slab, 0)

    return msg2


def _edge_prep(src, dst):
    perm = jnp.argsort(dst, stable=True)
    dsts = dst[perm]
    first = jnp.searchsorted(
        dsts, jnp.arange(NCHUNK + 1, dtype=jnp.int32) * CHUNK).astype(jnp.int32)
    cnt = first[1:] - first[:-1]
    pcnt = (cnt + 15) // 16 * 16
    sa = jnp.concatenate(
        [jnp.zeros((1,), jnp.int32), jnp.cumsum(pcnt).astype(jnp.int32)])
    ci = dsts // CHUNK
    pos = sa[ci] + (jnp.arange(E, dtype=jnp.int32) - first[ci])
    srcs_p = jnp.zeros((EP,), jnp.int32).at[pos].add(
        src[perm], unique_indices=True)
    dls_p = jnp.zeros((EP,), jnp.int32).at[pos].add(
        dsts % CHUNK, unique_indices=True)
    idx8 = jnp.arange(NCHUNK, dtype=jnp.int32) * 8
    s8 = jnp.zeros((NCHUNK * 8 + 16,), jnp.int32)
    s8 = s8.at[idx8].set(sa[:-1]).at[idx8 + 1].set(sa[1:])
    return perm, pos, srcs_p, dls_p, s8


def _batch_norm(x, gamma, beta):
    mu = x.mean(axis=0, keepdims=True)
    var = x.var(axis=0, keepdims=True)
    return (x - mu) / jnp.sqrt(var + 1e-5) * gamma + beta


def kernel(x, edge_index, W1, att_src1, att_dst1, b1, g1, be1,
           W2, att_src2, att_dst2, b2, g2, be2):
    src = edge_index[0].astype(jnp.int32)
    dst = edge_index[1].astype(jnp.int32)

    perm, pos, srcs_p, dls_p, s8 = _edge_prep(src, dst)
    sc_agg = _make_agg()

    h = (x @ W1).reshape(N, 2, HID)
    alpha_s = jnp.sum(h * att_src1[None, :, :], axis=-1)
    alpha_d = jnp.sum(h * att_dst1[None, :, :], axis=-1)
    e = jax.nn.leaky_relu(alpha_s[src] + alpha_d[dst], negative_slope=0.2)
    m = jax.ops.segment_max(e, dst, num_segments=N)
    ex = jnp.exp(e - m[dst])
    s = jax.ops.segment_sum(ex, dst, num_segments=N)
    alpha = ex / (s[dst] + 1e-16)
    w0_p = jnp.zeros((EP,), jnp.float32).at[pos].add(
        alpha[perm, 0], unique_indices=True)
    w1_p = jnp.zeros((EP,), jnp.float32).at[pos].add(
        alpha[perm, 1], unique_indices=True)
    out1 = sc_agg(h.reshape(N, D), srcs_p, dls_p, w0_p, w1_p, s8)[:N]
    h1 = out1 + b1
    h1 = _batch_norm(h1, g1, be1)
    h1 = jax.nn.relu(h1)

    h2 = (h1 @ W2).reshape(N, 1, HID)
    alpha_s2 = jnp.sum(h2 * att_src2[None, :, :], axis=-1)
    alpha_d2 = jnp.sum(h2 * att_dst2[None, :, :], axis=-1)
    e2 = jax.nn.leaky_relu(alpha_s2[src] + alpha_d2[dst], negative_slope=0.2)
    m2 = jax.ops.segment_max(e2, dst, num_segments=N)
    ex2 = jnp.exp(e2 - m2[dst])
    s2 = jax.ops.segment_sum(ex2, dst, num_segments=N)
    alpha2 = ex2 / (s2[dst] + 1e-16)
    src_pad = jnp.concatenate([src, jnp.zeros((2048,), jnp.int32)])
    w2_pad = jnp.concatenate([alpha2[:, 0], jnp.zeros((2048,), jnp.float32)])
    msg2 = _make_msg2()(h2.reshape(N, HID), src_pad, w2_pad)
    out2 = jax.ops.segment_sum(
        msg2.reshape(E, 1, HID), dst, num_segments=N)
    h2o = out2.mean(axis=1) + b2
    h2b = _batch_norm(h2o, g2, be2)
    return h2b.mean(axis=0, keepdims=True)
